# Initial kernel scaffold; baseline (speedup 1.0000x reference)
import jax
import jax.numpy as jnp
from jax import lax
from jax.experimental import pallas as pl
from jax.experimental.pallas import tpu as pltpu
from jax.experimental.pallas import tpu_sc as plsc

_NC = 2
_NS = 16
_NW = _NC * _NS
_C = 128


def _embed_body(idx_hbm, table_hbm, out_hbm, idx_v, rows_v, gsem):
    wid = lax.axis_index("s") * _NC + lax.axis_index("c")
    n_chunks = idx_hbm.shape[1]
    pltpu.sync_copy(idx_hbm.at[wid], idx_v)

    def body(j, carry):
        pltpu.async_copy(table_hbm.at[idx_v.at[j]], rows_v, gsem).wait()
        base = (wid * n_chunks + j) * _C
        pltpu.sync_copy(rows_v, out_hbm.at[pl.ds(base, _C)])
        return carry

    lax.fori_loop(0, n_chunks, body, 0)


def kernel(x, table):
    b0, b1 = x.shape
    vocab, dim = table.shape
    batch = b0 * b1
    n_chunks = batch // (_NW * _C)
    idx = x.reshape(_NW, n_chunks, _C).astype(jnp.int32)

    mesh = plsc.VectorSubcoreMesh(core_axis_name="c", subcore_axis_name="s")
    run = pl.kernel(
        _embed_body,
        out_type=jax.ShapeDtypeStruct((batch, dim), table.dtype),
        mesh=mesh,
        scratch_types=[
            pltpu.VMEM((n_chunks, _C), jnp.int32),
            pltpu.VMEM((_C, dim), jnp.float32),
            pltpu.SemaphoreType.DMA,
        ],
    )
    out = run(idx, table)
    return out.reshape(b0, b1, dim)


# speedup vs baseline: 1.6856x; 1.6856x over previous
import jax
import jax.numpy as jnp
from jax import lax
from jax.experimental import pallas as pl
from jax.experimental.pallas import tpu as pltpu
from jax.experimental.pallas import tpu_sc as plsc

_NC = 2
_NS = 16
_NW = _NC * _NS
_C = 128


def _embed_body(idx_hbm, table_hbm, out_hbm, idx_v, rows_v, gsem):
    wid = lax.axis_index("s") * _NC + lax.axis_index("c")
    n_chunks = idx_hbm.shape[1]
    pltpu.sync_copy(idx_hbm.at[wid], idx_v)

    def body(j, carry):
        pltpu.async_copy(table_hbm.at[idx_v.at[j]], rows_v, gsem).wait()
        base = (wid * n_chunks + j) * _C
        pltpu.sync_copy(rows_v, out_hbm.at[pl.ds(base, _C)])
        return carry

    lax.fori_loop(0, n_chunks, body, 0)


def kernel(x, table):
    b0, b1 = x.shape
    vocab, dim = table.shape
    batch = b0 * b1
    n_chunks = batch // (_NW * _C)
    idx = x.reshape(_NW, n_chunks, _C).astype(jnp.int32)

    mesh = plsc.VectorSubcoreMesh(core_axis_name="c", subcore_axis_name="s")
    run = pl.kernel(
        _embed_body,
        out_type=jax.ShapeDtypeStruct((batch, dim), table.dtype),
        mesh=mesh,
        scratch_types=[
            pltpu.VMEM((n_chunks, _C), jnp.int32),
            pltpu.VMEM((_C, dim), jnp.float32),
            pltpu.SemaphoreType.DMA,
        ],
        compiler_params=pltpu.CompilerParams(use_tc_tiling_on_sc=False),
    )
    out = run(idx, table)
    return out.reshape(b0, b1, dim)


# speedup vs baseline: 1.8712x; 1.1101x over previous
import jax
import jax.numpy as jnp
from jax import lax
from jax.experimental import pallas as pl
from jax.experimental.pallas import tpu as pltpu
from jax.experimental.pallas import tpu_sc as plsc

_NC = 2
_NS = 16
_NW = _NC * _NS
_C = 512


def _embed_body(idx_hbm, table_hbm, out_hbm, idx_v, bufs, gsem, ssem):
    wid = lax.axis_index("s") * _NC + lax.axis_index("c")
    n = idx_hbm.shape[1]
    dim = table_hbm.shape[1]
    pltpu.sync_copy(idx_hbm.at[wid], idx_v)
    pltpu.async_copy(table_hbm.at[idx_v.at[0]], bufs.at[0], gsem)

    def body(j, carry):
        slot = lax.rem(j, 2)
        nslot = lax.rem(j + 1, 2)
        pltpu.make_async_copy(table_hbm.at[idx_v.at[j]], bufs.at[slot], gsem).wait()
        base = (wid * n + j) * _C
        pltpu.async_copy(bufs.at[slot], out_hbm.at[pl.ds(base, _C)], ssem)

        @pl.when(j >= 1)
        def _():
            pltpu.make_async_copy(bufs.at[nslot], out_hbm.at[pl.ds(0, _C)], ssem).wait()

        @pl.when(j + 1 < n)
        def _():
            pltpu.async_copy(table_hbm.at[idx_v.at[j + 1]], bufs.at[nslot], gsem)

        return carry

    lax.fori_loop(0, n, body, 0)
    pltpu.make_async_copy(bufs.at[0], out_hbm.at[pl.ds(0, _C)], ssem).wait()


def kernel(x, table):
    b0, b1 = x.shape
    vocab, dim = table.shape
    batch = b0 * b1
    n_chunks = batch // (_NW * _C)
    idx = x.reshape(_NW, n_chunks, _C).astype(jnp.int32)

    mesh = plsc.VectorSubcoreMesh(core_axis_name="c", subcore_axis_name="s")
    run = pl.kernel(
        _embed_body,
        out_type=jax.ShapeDtypeStruct((batch, dim), table.dtype),
        mesh=mesh,
        scratch_types=[
            pltpu.VMEM((n_chunks, _C), jnp.int32),
            pltpu.VMEM((2, _C, dim), jnp.float32),
            pltpu.SemaphoreType.DMA,
            pltpu.SemaphoreType.DMA,
        ],
        compiler_params=pltpu.CompilerParams(use_tc_tiling_on_sc=False),
    )
    out = run(idx, table)
    return out.reshape(b0, b1, dim)
